# baseline (speedup 1.0000x reference)
import functools

import jax
import jax.numpy as jnp
from jax.experimental import pallas as pl
from jax.experimental.pallas import tpu as pltpu

B = 16
A = 2048
C = 1024
BA = 256
NJ = A // BA


def _ce_kernel(lens_ref, a_ref, y_ref, o_ref):
    b = pl.program_id(0)
    j = pl.program_id(1)
    length = lens_ref[b]
    start = j * BA

    @pl.when(start < length)
    def _compute():
        x = y_ref[0]
        m = jnp.max(x, axis=1, keepdims=True)
        e = jnp.exp(x - m)
        s = jnp.sum(e, axis=1, keepdims=True)
        logz = m + jnp.log(s)
        aa = a_ref[0, 0]
        cols = jax.lax.broadcasted_iota(jnp.int32, (BA, C), 1)
        tl = jnp.sum(jnp.where(cols == aa, x, 0.0), axis=1, keepdims=True)
        pos = start + jax.lax.broadcasted_iota(jnp.int32, (BA, 1), 0)
        valid = pos < length
        o_ref[0, 0] = jnp.where(valid, logz - tl, 0.0)

    @pl.when(start >= length)
    def _zero():
        o_ref[0, 0] = jnp.zeros((BA, 1), jnp.float32)


def _clamped_block(b, j, lens):
    length = lens[b]
    jlast = jnp.maximum((length + BA - 1) // BA - 1, 0)
    return jnp.minimum(j, jlast)


def _y_map(b, j, lens):
    return (b, _clamped_block(b, j, lens), 0)


def _a_map(b, j, lens):
    return (b, _clamped_block(b, j, lens), 0, 0)


def _o_map(b, j, lens):
    return (b, j, 0, 0)


@jax.jit
def kernel(y_true, y_pred, lengths):
    args = y_true.reshape(B, NJ, BA, 1).astype(jnp.int32)
    lens = lengths.astype(jnp.int32)
    out = pl.pallas_call(
        _ce_kernel,
        grid_spec=pltpu.PrefetchScalarGridSpec(
            num_scalar_prefetch=1,
            grid=(B, NJ),
            in_specs=[
                pl.BlockSpec((1, 1, BA, 1), _a_map),
                pl.BlockSpec((1, BA, C), _y_map),
            ],
            out_specs=pl.BlockSpec((1, 1, BA, 1), _o_map),
        ),
        out_shape=jax.ShapeDtypeStruct((B, NJ, BA, 1), jnp.float32),
    )(lens, args, y_pred)
    return out.reshape(B, A)
